# speedup vs baseline: 1.1415x; 1.1415x over previous
import functools

import jax
import jax.numpy as jnp
from jax import lax
from jax.experimental import pallas as pl
from jax.experimental.pallas import tpu as pltpu

N = 10000
E = 320000
D = 128
L = 5
G = 256
FEATS = 9
ATOM_VOCAB = 120
BOND_VOCAB = 5

NP = 10240
BLK = 512
NBLK = NP // BLK


def _enc_body(x_ref, emb_ref, o_ref):
    x = x_ref[...]
    acc = jnp.zeros((BLK, D), dtype=jnp.float32)
    for f in range(FEATS):
        col = x[:, f][:, None]
        iota = lax.broadcasted_iota(jnp.int32, (1, ATOM_VOCAB), 1)
        oh = (col == iota).astype(jnp.float32)
        acc = acc + jax.lax.dot(oh, emb_ref[f], preferred_element_type=jnp.float32)
    o_ref[...] = acc


@jax.jit
def _encoder(xp, atom_emb):
    return pl.pallas_call(
        _enc_body,
        grid=(NBLK,),
        in_specs=[
            pl.BlockSpec((BLK, FEATS), lambda i: (i, 0)),
            pl.BlockSpec((FEATS, ATOM_VOCAB, D), lambda i: (0, 0, 0)),
        ],
        out_specs=pl.BlockSpec((BLK, D), lambda i: (i, 0)),
        out_shape=jax.ShapeDtypeStruct((NP, D), jnp.float32),
    )(xp, atom_emb)


def _mlp_pool_body(h_ref, a_ref, bat_ref, eps_ref, w1_ref, b1_ref, w2_ref,
                   b2_ref, ho_ref, pool_ref):
    eps1 = 1.0 + eps_ref[0, 0]
    z = eps1 * h_ref[...] + a_ref[0] + a_ref[1]
    t = jax.lax.dot(z, w1_ref[...], preferred_element_type=jnp.float32) + b1_ref[...]
    t = jnp.maximum(t, 0.0)
    h2 = jax.lax.dot(t, w2_ref[...], preferred_element_type=jnp.float32) + b2_ref[...]
    ho_ref[...] = h2
    col = bat_ref[...][:, 0][:, None]
    iota = lax.broadcasted_iota(jnp.int32, (1, G), 1)
    oh = (col == iota).astype(jnp.float32)
    p = lax.dot_general(oh, h2, (((0,), (0,)), ((), ())),
                        preferred_element_type=jnp.float32)

    @pl.when(pl.program_id(0) == 0)
    def _init():
        pool_ref[...] = p

    @pl.when(pl.program_id(0) != 0)
    def _acc():
        pool_ref[...] = pool_ref[...] + p


@jax.jit
def _mlp_pool(h, aggr2, batp, eps_l, W1l, b1l, W2l, b2l):
    return pl.pallas_call(
        _mlp_pool_body,
        grid=(NBLK,),
        in_specs=[
            pl.BlockSpec((BLK, D), lambda i: (i, 0)),
            pl.BlockSpec((2, BLK, D), lambda i: (0, i, 0)),
            pl.BlockSpec((BLK, 1), lambda i: (i, 0)),
            pl.BlockSpec(memory_space=pltpu.SMEM),
            pl.BlockSpec((D, 2 * D), lambda i: (0, 0)),
            pl.BlockSpec((1, 2 * D), lambda i: (0, 0)),
            pl.BlockSpec((2 * D, D), lambda i: (0, 0)),
            pl.BlockSpec((1, D), lambda i: (0, 0)),
        ],
        out_specs=[
            pl.BlockSpec((BLK, D), lambda i: (i, 0)),
            pl.BlockSpec((G, D), lambda i: (0, 0)),
        ],
        out_shape=[
            jax.ShapeDtypeStruct((NP, D), jnp.float32),
            jax.ShapeDtypeStruct((G, D), jnp.float32),
        ],
    )(h, aggr2, batp, eps_l, W1l, b1l, W2l, b2l)


def kernel(x, edge_index, edge_attr, batch, atom_emb, bond_emb, W1, b1, W2, b2, eps):
    xp = jnp.zeros((NP, FEATS), jnp.int32).at[:N].set(x.astype(jnp.int32))
    batp = jnp.full((NP, 1), G, jnp.int32).at[:N, 0].set(batch.astype(jnp.int32))
    src = edge_index[0].astype(jnp.int32)
    dst = edge_index[1].astype(jnp.int32)
    bidx = jnp.clip(edge_attr[:, 0], 0, BOND_VOCAB - 1).astype(jnp.int32)
    e = jnp.take(bond_emb, bidx, axis=0)

    h = _encoder(xp, atom_emb)
    outs = []
    pools = []
    for l in range(L):
        m = jax.nn.relu(jnp.take(h, src, axis=0) + e)
        aggr = jax.ops.segment_sum(m, dst, num_segments=NP)
        aggr2 = jnp.stack([aggr, jnp.zeros_like(aggr)])
        eps_l = eps[l].reshape(1, 1)
        h, pool_l = _mlp_pool(h, aggr2, batp, eps_l, W1[l],
                              b1[l].reshape(1, -1), W2[l], b2[l].reshape(1, -1))
        outs.append(h[:N])
        pools.append(pool_l)
    node_embs = jnp.concatenate(outs, axis=-1)
    graph_embs = jnp.concatenate(pools, axis=-1)
    return graph_embs, node_embs


# speedup vs baseline: 2.1635x; 1.8953x over previous
import functools

import jax
import jax.numpy as jnp
from jax import lax
from jax.experimental import pallas as pl
from jax.experimental.pallas import tpu as pltpu
from jax.experimental.pallas import tpu_sc as plsc

N = 10000
E = 320000
D = 128
L = 5
G = 256
FEATS = 9
ATOM_VOCAB = 120
BOND_VOCAB = 5

NP = 10240
BLK = 512
NBLK = NP // BLK

NC = 2
NS = 16
NW = NC * NS
C = 128
CHUNKS_PW = 79
EPW = CHUNKS_PW * C
EP = NW * EPW
ROWS_PT = NP // NS
_SC_MESH = plsc.VectorSubcoreMesh(core_axis_name="c", subcore_axis_name="s")


def _edge_body(h_hbm, src_hbm, dst_hbm, bidx_hbm, bond_hbm, out_hbm,
               src_v, dst_v, bidx_v, rows_v, bond_v, aggr_sh, sem):
    c = lax.axis_index("c")
    s = lax.axis_index("s")
    wid = s * NC + c

    zero = jnp.zeros((16,), jnp.float32)

    def _zrow(i, carry):
        for j in range(D // 16):
            rows_v[i, pl.ds(j * 16, 16)] = zero
        return carry

    lax.fori_loop(0, C, _zrow, 0)
    for k in range(ROWS_PT // C):
        pltpu.sync_copy(rows_v, aggr_sh.at[pl.ds(s * ROWS_PT + k * C, C)])
    plsc.subcore_barrier()

    pltpu.sync_copy(bond_hbm, bond_v)

    def _chunk(k, carry):
        off = wid * EPW + k * C
        pltpu.sync_copy(src_hbm.at[pl.ds(off, C)], src_v)
        pltpu.sync_copy(dst_hbm.at[pl.ds(off, C)], dst_v)
        pltpu.sync_copy(bidx_hbm.at[pl.ds(off, C)], bidx_v)
        pltpu.async_copy(h_hbm.at[src_v], rows_v, sem).wait()

        def _grp(g, cc):
            bvec = bidx_v[pl.ds(g * 16, 16)]
            for i16 in range(16):
                b = bvec[i16]
                row = g * 16 + i16
                for j in range(D // 16):
                    sl = pl.ds(j * 16, 16)
                    rows_v[row, sl] = jnp.maximum(
                        rows_v[row, sl] + bond_v[b, sl], 0.0)
            return cc

        lax.fori_loop(0, C // 16, _grp, 0)
        pltpu.sync_copy(rows_v, aggr_sh.at[dst_v], add=True)
        return carry

    lax.fori_loop(0, CHUNKS_PW, _chunk, 0)

    plsc.subcore_barrier()
    for k in range(ROWS_PT // C):
        start = s * ROWS_PT + k * C
        pltpu.sync_copy(aggr_sh.at[pl.ds(start, C)],
                        out_hbm.at[c, pl.ds(start, C)])


@jax.jit
def _edge_aggr(h, src_p, dst_p, bidx_p, bond_ext):
    fn = functools.partial(
        pl.kernel,
        mesh=_SC_MESH,
        out_type=jax.ShapeDtypeStruct((NC, NP, D), jnp.float32),
        scratch_types=[
            pltpu.VMEM((C,), jnp.int32),
            pltpu.VMEM((C,), jnp.int32),
            pltpu.VMEM((C,), jnp.int32),
            pltpu.VMEM((C, D), jnp.float32),
            pltpu.VMEM((8, D), jnp.float32),
            pltpu.VMEM_SHARED((NP, D), jnp.float32),
            pltpu.SemaphoreType.DMA,
        ],
    )(_edge_body)
    return fn(h, src_p, dst_p, bidx_p, bond_ext)


def _enc_body(x_ref, emb_ref, o_ref):
    x = x_ref[...]
    acc = jnp.zeros((BLK, D), dtype=jnp.float32)
    for f in range(FEATS):
        col = x[:, f][:, None]
        iota = lax.broadcasted_iota(jnp.int32, (1, ATOM_VOCAB), 1)
        oh = (col == iota).astype(jnp.float32)
        acc = acc + jax.lax.dot(oh, emb_ref[f], preferred_element_type=jnp.float32)
    o_ref[...] = acc


@jax.jit
def _encoder(xp, atom_emb):
    return pl.pallas_call(
        _enc_body,
        grid=(NBLK,),
        in_specs=[
            pl.BlockSpec((BLK, FEATS), lambda i: (i, 0)),
            pl.BlockSpec((FEATS, ATOM_VOCAB, D), lambda i: (0, 0, 0)),
        ],
        out_specs=pl.BlockSpec((BLK, D), lambda i: (i, 0)),
        out_shape=jax.ShapeDtypeStruct((NP, D), jnp.float32),
    )(xp, atom_emb)


def _mlp_pool_body(h_ref, a_ref, bat_ref, eps_ref, w1_ref, b1_ref, w2_ref,
                   b2_ref, ho_ref, pool_ref):
    eps1 = 1.0 + eps_ref[0, 0]
    z = eps1 * h_ref[...] + a_ref[0] + a_ref[1]
    t = jax.lax.dot(z, w1_ref[...], preferred_element_type=jnp.float32) + b1_ref[...]
    t = jnp.maximum(t, 0.0)
    h2 = jax.lax.dot(t, w2_ref[...], preferred_element_type=jnp.float32) + b2_ref[...]
    ho_ref[...] = h2
    col = bat_ref[...][:, 0][:, None]
    iota = lax.broadcasted_iota(jnp.int32, (1, G), 1)
    oh = (col == iota).astype(jnp.float32)
    p = lax.dot_general(oh, h2, (((0,), (0,)), ((), ())),
                        preferred_element_type=jnp.float32)

    @pl.when(pl.program_id(0) == 0)
    def _init():
        pool_ref[...] = p

    @pl.when(pl.program_id(0) != 0)
    def _acc():
        pool_ref[...] = pool_ref[...] + p


@jax.jit
def _mlp_pool(h, aggr2, batp, eps_l, W1l, b1l, W2l, b2l):
    return pl.pallas_call(
        _mlp_pool_body,
        grid=(NBLK,),
        in_specs=[
            pl.BlockSpec((BLK, D), lambda i: (i, 0)),
            pl.BlockSpec((2, BLK, D), lambda i: (0, i, 0)),
            pl.BlockSpec((BLK, 1), lambda i: (i, 0)),
            pl.BlockSpec(memory_space=pltpu.SMEM),
            pl.BlockSpec((D, 2 * D), lambda i: (0, 0)),
            pl.BlockSpec((1, 2 * D), lambda i: (0, 0)),
            pl.BlockSpec((2 * D, D), lambda i: (0, 0)),
            pl.BlockSpec((1, D), lambda i: (0, 0)),
        ],
        out_specs=[
            pl.BlockSpec((BLK, D), lambda i: (i, 0)),
            pl.BlockSpec((G, D), lambda i: (0, 0)),
        ],
        out_shape=[
            jax.ShapeDtypeStruct((NP, D), jnp.float32),
            jax.ShapeDtypeStruct((G, D), jnp.float32),
        ],
    )(h, aggr2, batp, eps_l, W1l, b1l, W2l, b2l)


def kernel(x, edge_index, edge_attr, batch, atom_emb, bond_emb, W1, b1, W2, b2, eps):
    xp = jnp.zeros((NP, FEATS), jnp.int32).at[:N].set(x.astype(jnp.int32))
    batp = jnp.full((NP, 1), G, jnp.int32).at[:N, 0].set(batch.astype(jnp.int32))
    src_p = jnp.full((EP,), N, jnp.int32).at[:E].set(edge_index[0].astype(jnp.int32))
    dst_p = jnp.full((EP,), N, jnp.int32).at[:E].set(edge_index[1].astype(jnp.int32))
    bidx = jnp.clip(edge_attr[:, 0], 0, BOND_VOCAB - 1).astype(jnp.int32)
    bidx_p = jnp.full((EP,), BOND_VOCAB, jnp.int32).at[:E].set(bidx)
    bond_ext = jnp.zeros((8, D), jnp.float32).at[:BOND_VOCAB].set(bond_emb)

    h = _encoder(xp, atom_emb)
    outs = []
    pools = []
    for l in range(L):
        aggr2 = _edge_aggr(h, src_p, dst_p, bidx_p, bond_ext)
        eps_l = eps[l].reshape(1, 1)
        h, pool_l = _mlp_pool(h, aggr2, batp, eps_l, W1[l],
                              b1[l].reshape(1, -1), W2[l], b2[l].reshape(1, -1))
        outs.append(h[:N])
        pools.append(pool_l)
    node_embs = jnp.concatenate(outs, axis=-1)
    graph_embs = jnp.concatenate(pools, axis=-1)
    return graph_embs, node_embs


# speedup vs baseline: 3.0522x; 1.4107x over previous
import functools

import jax
import jax.numpy as jnp
from jax import lax
from jax.experimental import pallas as pl
from jax.experimental.pallas import tpu as pltpu
from jax.experimental.pallas import tpu_sc as plsc

N = 10000
E = 320000
D = 128
L = 5
G = 256
FEATS = 9
ATOM_VOCAB = 120
BOND_VOCAB = 5

NP = 10240
BLK = 512
NBLK = NP // BLK

NC = 2
NS = 16
NW = NC * NS
C = 96
CHUNKS_PW = 105
EPW = CHUNKS_PW * C
EP = NW * EPW
ROWS_PT = NP // NS
_SC_MESH = plsc.VectorSubcoreMesh(core_axis_name="c", subcore_axis_name="s")


def _edge_body(h_hbm, idx_hbm, bond_hbm, out_hbm,
               i0, i1, i2, r0, r1, r2, bond_v, aggr_sh, sem_g, sem_s):
    c = lax.axis_index("c")
    s = lax.axis_index("s")
    wid = s * NC + c
    base = wid * CHUNKS_PW
    idxs = (i0, i1, i2)
    rows = (r0, r1, r2)

    zero = jnp.zeros((16,), jnp.float32)

    def _zrow(i, carry):
        for j in range(D // 16):
            r0[i, pl.ds(j * 16, 16)] = zero
        return carry

    lax.fori_loop(0, C, _zrow, 0)
    _nfull, _rem = divmod(ROWS_PT, C)
    for k in range(_nfull):
        pltpu.sync_copy(r0, aggr_sh.at[pl.ds(s * ROWS_PT + k * C, C)])
    if _rem:
        pltpu.sync_copy(r0.at[pl.ds(0, _rem)],
                        aggr_sh.at[pl.ds(s * ROWS_PT + _nfull * C, _rem)])
    plsc.subcore_barrier()

    pltpu.sync_copy(bond_hbm, bond_v)

    def _fetch(k, t):
        pltpu.sync_copy(idx_hbm.at[base + k], idxs[t])
        pltpu.async_copy(h_hbm.at[idxs[t].at[0]], rows[t], sem_g)

    def _drain(sem, t):
        pltpu.make_async_copy(h_hbm.at[pl.ds(0, C)], rows[t], sem).wait()

    def _compute(t):
        rv = rows[t]
        iv = idxs[t]

        def _grp(g2, cc):
            bvec = iv[2, pl.ds(g2 * 16, 16)]
            for i16 in range(16):
                b = bvec[i16]
                ro = g2 * 16 + i16
                for j in range(D // 16):
                    sl = pl.ds(j * 16, 16)
                    rv[ro, sl] = jnp.maximum(rv[ro, sl] + bond_v[b, sl], 0.0)
            return cc

        lax.fori_loop(0, C // 16, _grp, 0)

    def _scat(t):
        pltpu.async_copy(rows[t], aggr_sh.at[idxs[t].at[1]], sem_s, add=True)

    _fetch(0, 0)
    G3 = CHUNKS_PW // 3

    def _iter(g, cc):
        k0 = g * 3

        @pl.when(g >= 1)
        def _():
            _drain(sem_s, 1)

        _fetch(k0 + 1, 1)
        _drain(sem_g, 0)
        _compute(0)
        _scat(0)

        @pl.when(g >= 1)
        def _():
            _drain(sem_s, 2)

        _fetch(k0 + 2, 2)
        _drain(sem_g, 1)
        _compute(1)
        _scat(1)

        @pl.when(g < G3 - 1)
        def _():
            _drain(sem_s, 0)
            _fetch(k0 + 3, 0)

        _drain(sem_g, 2)
        _compute(2)
        _scat(2)
        return cc

    lax.fori_loop(0, G3, _iter, 0)
    _drain(sem_s, 0)
    _drain(sem_s, 1)
    _drain(sem_s, 2)

    plsc.subcore_barrier()
    for k in range(_nfull):
        start = s * ROWS_PT + k * C
        pltpu.sync_copy(aggr_sh.at[pl.ds(start, C)],
                        out_hbm.at[c, pl.ds(start, C)])
    if _rem:
        start = s * ROWS_PT + _nfull * C
        pltpu.sync_copy(aggr_sh.at[pl.ds(start, _rem)],
                        out_hbm.at[c, pl.ds(start, _rem)])


@jax.jit
def _edge_aggr(h, idx_p, bond_ext):
    fn = functools.partial(
        pl.kernel,
        mesh=_SC_MESH,
        out_type=jax.ShapeDtypeStruct((NC, NP, D), jnp.float32),
        scratch_types=[
            pltpu.VMEM((3, C), jnp.int32),
            pltpu.VMEM((3, C), jnp.int32),
            pltpu.VMEM((3, C), jnp.int32),
            pltpu.VMEM((C, D), jnp.float32),
            pltpu.VMEM((C, D), jnp.float32),
            pltpu.VMEM((C, D), jnp.float32),
            pltpu.VMEM((8, D), jnp.float32),
            pltpu.VMEM_SHARED((NP, D), jnp.float32),
            pltpu.SemaphoreType.DMA,
            pltpu.SemaphoreType.DMA,
        ],
    )(_edge_body)
    return fn(h, idx_p, bond_ext)


def _enc_body(x_ref, emb_ref, o_ref):
    x = x_ref[...]
    acc = jnp.zeros((BLK, D), dtype=jnp.float32)
    for f in range(FEATS):
        col = x[:, f][:, None]
        iota = lax.broadcasted_iota(jnp.int32, (1, ATOM_VOCAB), 1)
        oh = (col == iota).astype(jnp.float32)
        acc = acc + jax.lax.dot(oh, emb_ref[f], preferred_element_type=jnp.float32)
    o_ref[...] = acc


@jax.jit
def _encoder(xp, atom_emb):
    return pl.pallas_call(
        _enc_body,
        grid=(NBLK,),
        in_specs=[
            pl.BlockSpec((BLK, FEATS), lambda i: (i, 0)),
            pl.BlockSpec((FEATS, ATOM_VOCAB, D), lambda i: (0, 0, 0)),
        ],
        out_specs=pl.BlockSpec((BLK, D), lambda i: (i, 0)),
        out_shape=jax.ShapeDtypeStruct((NP, D), jnp.float32),
    )(xp, atom_emb)


def _mlp_pool_body(h_ref, a_ref, bat_ref, eps_ref, w1_ref, b1_ref, w2_ref,
                   b2_ref, ho_ref, pool_ref):
    eps1 = 1.0 + eps_ref[0, 0]
    z = eps1 * h_ref[...] + a_ref[0] + a_ref[1]
    t = jax.lax.dot(z, w1_ref[...], preferred_element_type=jnp.float32) + b1_ref[...]
    t = jnp.maximum(t, 0.0)
    h2 = jax.lax.dot(t, w2_ref[...], preferred_element_type=jnp.float32) + b2_ref[...]
    ho_ref[...] = h2
    col = bat_ref[...][:, 0][:, None]
    iota = lax.broadcasted_iota(jnp.int32, (1, G), 1)
    oh = (col == iota).astype(jnp.float32)
    p = lax.dot_general(oh, h2, (((0,), (0,)), ((), ())),
                        preferred_element_type=jnp.float32)

    @pl.when(pl.program_id(0) == 0)
    def _init():
        pool_ref[...] = p

    @pl.when(pl.program_id(0) != 0)
    def _acc():
        pool_ref[...] = pool_ref[...] + p


@jax.jit
def _mlp_pool(h, aggr2, batp, eps_l, W1l, b1l, W2l, b2l):
    return pl.pallas_call(
        _mlp_pool_body,
        grid=(NBLK,),
        in_specs=[
            pl.BlockSpec((BLK, D), lambda i: (i, 0)),
            pl.BlockSpec((2, BLK, D), lambda i: (0, i, 0)),
            pl.BlockSpec((BLK, 1), lambda i: (i, 0)),
            pl.BlockSpec(memory_space=pltpu.SMEM),
            pl.BlockSpec((D, 2 * D), lambda i: (0, 0)),
            pl.BlockSpec((1, 2 * D), lambda i: (0, 0)),
            pl.BlockSpec((2 * D, D), lambda i: (0, 0)),
            pl.BlockSpec((1, D), lambda i: (0, 0)),
        ],
        out_specs=[
            pl.BlockSpec((BLK, D), lambda i: (i, 0)),
            pl.BlockSpec((G, D), lambda i: (0, 0)),
        ],
        out_shape=[
            jax.ShapeDtypeStruct((NP, D), jnp.float32),
            jax.ShapeDtypeStruct((G, D), jnp.float32),
        ],
    )(h, aggr2, batp, eps_l, W1l, b1l, W2l, b2l)


def kernel(x, edge_index, edge_attr, batch, atom_emb, bond_emb, W1, b1, W2, b2, eps):
    xp = jnp.zeros((NP, FEATS), jnp.int32).at[:N].set(x.astype(jnp.int32))
    batp = jnp.full((NP, 1), G, jnp.int32).at[:N, 0].set(batch.astype(jnp.int32))
    bidx = jnp.clip(edge_attr[:, 0], 0, BOND_VOCAB - 1).astype(jnp.int32)
    idx_p = jnp.full((3, EP), N, jnp.int32)
    idx_p = idx_p.at[0, :E].set(edge_index[0].astype(jnp.int32))
    idx_p = idx_p.at[1, :E].set(edge_index[1].astype(jnp.int32))
    idx_p = idx_p.at[2, :E].set(bidx).at[2, E:].set(BOND_VOCAB)
    idx_p = jnp.transpose(idx_p.reshape(3, NW * CHUNKS_PW, C), (1, 0, 2))
    bond_ext = jnp.zeros((8, D), jnp.float32).at[:BOND_VOCAB].set(bond_emb)

    h = _encoder(xp, atom_emb)
    outs = []
    pools = []
    for l in range(L):
        aggr2 = _edge_aggr(h, idx_p, bond_ext)
        eps_l = eps[l].reshape(1, 1)
        h, pool_l = _mlp_pool(h, aggr2, batp, eps_l, W1[l],
                              b1[l].reshape(1, -1), W2[l], b2[l].reshape(1, -1))
        outs.append(h[:N])
        pools.append(pool_l)
    node_embs = jnp.concatenate(outs, axis=-1)
    graph_embs = jnp.concatenate(pools, axis=-1)
    return graph_embs, node_embs


# speedup vs baseline: 3.0682x; 1.0053x over previous
import functools

import jax
import jax.numpy as jnp
from jax import lax
from jax.experimental import pallas as pl
from jax.experimental.pallas import tpu as pltpu
from jax.experimental.pallas import tpu_sc as plsc

N = 10000
E = 320000
D = 128
L = 5
G = 256
FEATS = 9
ATOM_VOCAB = 120
BOND_VOCAB = 5

NP = 10240
BLK = 512
NBLK = NP // BLK

NC = 2
NS = 16
NW = NC * NS
C = 80
CHUNKS_PW = 128
EPW = CHUNKS_PW * C
EP = NW * EPW
ROWS_PT = NP // NS
_SC_MESH = plsc.VectorSubcoreMesh(core_axis_name="c", subcore_axis_name="s")


def _edge_body(h_hbm, idx_hbm, bond_hbm, out_hbm,
               i0, i1, i2, i3, r0, r1, r2, r3, bond_v, aggr_sh,
               sem_i, sem_g, sem_s):
    c = lax.axis_index("c")
    s = lax.axis_index("s")
    wid = s * NC + c
    base = wid * CHUNKS_PW
    idxs = (i0, i1, i2, i3)
    rows = (r0, r1, r2, r3)

    zero = jnp.zeros((16,), jnp.float32)

    def _zrow(i, carry):
        for j in range(D // 16):
            r0[i, pl.ds(j * 16, 16)] = zero
        return carry

    lax.fori_loop(0, C, _zrow, 0)
    _nfull, _rem = divmod(ROWS_PT, C)
    for k in range(_nfull):
        pltpu.sync_copy(r0, aggr_sh.at[pl.ds(s * ROWS_PT + k * C, C)])
    if _rem:
        pltpu.sync_copy(r0.at[pl.ds(0, _rem)],
                        aggr_sh.at[pl.ds(s * ROWS_PT + _nfull * C, _rem)])
    plsc.subcore_barrier()

    pltpu.sync_copy(bond_hbm, bond_v)

    def _fetch_idx(k, t):
        pltpu.async_copy(idx_hbm.at[base + k], idxs[t], sem_i)

    def _gather(t):
        pltpu.async_copy(h_hbm.at[idxs[t].at[0]], rows[t], sem_g)

    def _drain_rows(sem, t):
        pltpu.make_async_copy(h_hbm.at[pl.ds(0, C)], rows[t], sem).wait()

    def _drain_idx(t):
        pltpu.make_async_copy(idx_hbm.at[0], idxs[t], sem_i).wait()

    def _compute(t):
        rv = rows[t]
        iv = idxs[t]

        def _grp(g2, cc):
            bvec = iv[2, pl.ds(g2 * 16, 16)]
            for i16 in range(16):
                b = bvec[i16]
                ro = g2 * 16 + i16
                for j in range(D // 16):
                    sl = pl.ds(j * 16, 16)
                    rv[ro, sl] = jnp.maximum(rv[ro, sl] + bond_v[b, sl], 0.0)
            return cc

        lax.fori_loop(0, C // 16, _grp, 0)

    def _scat(t):
        pltpu.async_copy(rows[t], aggr_sh.at[idxs[t].at[1]], sem_s, add=True)

    K = CHUNKS_PW
    _fetch_idx(0, 0)
    _fetch_idx(1, 1)
    _drain_idx(0)
    _gather(0)

    def _iter(g, cc):
        for j in range(4):
            kk = g * 4 + j
            t = j
            t1 = (j + 1) % 4
            t2 = (j + 2) % 4

            @pl.when(jnp.logical_and(kk >= 2, kk + 2 < K))
            def _():
                _drain_rows(sem_s, t2)

            @pl.when(kk + 2 < K)
            def _():
                _fetch_idx(kk + 2, t2)

            @pl.when(kk + 1 < K)
            def _():
                _drain_idx(t1)
                _gather(t1)

            _drain_rows(sem_g, t)
            _compute(t)
            _scat(t)
        return cc

    lax.fori_loop(0, K // 4, _iter, 0)
    for t in range(4):
        _drain_rows(sem_s, t)

    plsc.subcore_barrier()
    for k in range(_nfull):
        start = s * ROWS_PT + k * C
        pltpu.sync_copy(aggr_sh.at[pl.ds(start, C)],
                        out_hbm.at[c, pl.ds(start, C)])
    if _rem:
        start = s * ROWS_PT + _nfull * C
        pltpu.sync_copy(aggr_sh.at[pl.ds(start, _rem)],
                        out_hbm.at[c, pl.ds(start, _rem)])


@jax.jit
def _edge_aggr(h, idx_p, bond_ext):
    fn = functools.partial(
        pl.kernel,
        mesh=_SC_MESH,
        out_type=jax.ShapeDtypeStruct((NC, NP, D), jnp.float32),
        scratch_types=[
            pltpu.VMEM((3, C), jnp.int32),
            pltpu.VMEM((3, C), jnp.int32),
            pltpu.VMEM((3, C), jnp.int32),
            pltpu.VMEM((3, C), jnp.int32),
            pltpu.VMEM((C, D), jnp.float32),
            pltpu.VMEM((C, D), jnp.float32),
            pltpu.VMEM((C, D), jnp.float32),
            pltpu.VMEM((C, D), jnp.float32),
            pltpu.VMEM((8, D), jnp.float32),
            pltpu.VMEM_SHARED((NP, D), jnp.float32),
            pltpu.SemaphoreType.DMA,
            pltpu.SemaphoreType.DMA,
            pltpu.SemaphoreType.DMA,
        ],
    )(_edge_body)
    return fn(h, idx_p, bond_ext)


def _enc_body(x_ref, emb_ref, o_ref):
    x = x_ref[...]
    acc = jnp.zeros((BLK, D), dtype=jnp.float32)
    for f in range(FEATS):
        col = x[:, f][:, None]
        iota = lax.broadcasted_iota(jnp.int32, (1, ATOM_VOCAB), 1)
        oh = (col == iota).astype(jnp.float32)
        acc = acc + jax.lax.dot(oh, emb_ref[f], preferred_element_type=jnp.float32)
    o_ref[...] = acc


@jax.jit
def _encoder(xp, atom_emb):
    return pl.pallas_call(
        _enc_body,
        grid=(NBLK,),
        in_specs=[
            pl.BlockSpec((BLK, FEATS), lambda i: (i, 0)),
            pl.BlockSpec((FEATS, ATOM_VOCAB, D), lambda i: (0, 0, 0)),
        ],
        out_specs=pl.BlockSpec((BLK, D), lambda i: (i, 0)),
        out_shape=jax.ShapeDtypeStruct((NP, D), jnp.float32),
    )(xp, atom_emb)


def _mlp_pool_body(h_ref, a_ref, bat_ref, eps_ref, w1_ref, b1_ref, w2_ref,
                   b2_ref, ho_ref, pool_ref):
    eps1 = 1.0 + eps_ref[0, 0]
    z = eps1 * h_ref[...] + a_ref[0] + a_ref[1]
    t = jax.lax.dot(z, w1_ref[...], preferred_element_type=jnp.float32) + b1_ref[...]
    t = jnp.maximum(t, 0.0)
    h2 = jax.lax.dot(t, w2_ref[...], preferred_element_type=jnp.float32) + b2_ref[...]
    ho_ref[...] = h2
    col = bat_ref[...][:, 0][:, None]
    iota = lax.broadcasted_iota(jnp.int32, (1, G), 1)
    oh = (col == iota).astype(jnp.float32)
    p = lax.dot_general(oh, h2, (((0,), (0,)), ((), ())),
                        preferred_element_type=jnp.float32)

    @pl.when(pl.program_id(0) == 0)
    def _init():
        pool_ref[...] = p

    @pl.when(pl.program_id(0) != 0)
    def _acc():
        pool_ref[...] = pool_ref[...] + p


@jax.jit
def _mlp_pool(h, aggr2, batp, eps_l, W1l, b1l, W2l, b2l):
    return pl.pallas_call(
        _mlp_pool_body,
        grid=(NBLK,),
        in_specs=[
            pl.BlockSpec((BLK, D), lambda i: (i, 0)),
            pl.BlockSpec((2, BLK, D), lambda i: (0, i, 0)),
            pl.BlockSpec((BLK, 1), lambda i: (i, 0)),
            pl.BlockSpec(memory_space=pltpu.SMEM),
            pl.BlockSpec((D, 2 * D), lambda i: (0, 0)),
            pl.BlockSpec((1, 2 * D), lambda i: (0, 0)),
            pl.BlockSpec((2 * D, D), lambda i: (0, 0)),
            pl.BlockSpec((1, D), lambda i: (0, 0)),
        ],
        out_specs=[
            pl.BlockSpec((BLK, D), lambda i: (i, 0)),
            pl.BlockSpec((G, D), lambda i: (0, 0)),
        ],
        out_shape=[
            jax.ShapeDtypeStruct((NP, D), jnp.float32),
            jax.ShapeDtypeStruct((G, D), jnp.float32),
        ],
    )(h, aggr2, batp, eps_l, W1l, b1l, W2l, b2l)


def kernel(x, edge_index, edge_attr, batch, atom_emb, bond_emb, W1, b1, W2, b2, eps):
    xp = jnp.zeros((NP, FEATS), jnp.int32).at[:N].set(x.astype(jnp.int32))
    batp = jnp.full((NP, 1), G, jnp.int32).at[:N, 0].set(batch.astype(jnp.int32))
    bidx = jnp.clip(edge_attr[:, 0], 0, BOND_VOCAB - 1).astype(jnp.int32)
    idx_p = jnp.full((3, EP), N, jnp.int32)
    idx_p = idx_p.at[0, :E].set(edge_index[0].astype(jnp.int32))
    idx_p = idx_p.at[1, :E].set(edge_index[1].astype(jnp.int32))
    idx_p = idx_p.at[2, :E].set(bidx).at[2, E:].set(BOND_VOCAB)
    idx_p = jnp.transpose(idx_p.reshape(3, NW * CHUNKS_PW, C), (1, 0, 2))
    bond_ext = jnp.zeros((8, D), jnp.float32).at[:BOND_VOCAB].set(bond_emb)

    h = _encoder(xp, atom_emb)
    outs = []
    pools = []
    for l in range(L):
        aggr2 = _edge_aggr(h, idx_p, bond_ext)
        eps_l = eps[l].reshape(1, 1)
        h, pool_l = _mlp_pool(h, aggr2, batp, eps_l, W1[l],
                              b1[l].reshape(1, -1), W2[l], b2[l].reshape(1, -1))
        outs.append(h[:N])
        pools.append(pool_l)
    node_embs = jnp.concatenate(outs, axis=-1)
    graph_embs = jnp.concatenate(pools, axis=-1)
    return graph_embs, node_embs


# speedup vs baseline: 3.0719x; 1.0012x over previous
import functools

import jax
import jax.numpy as jnp
from jax import lax
from jax.experimental import pallas as pl
from jax.experimental.pallas import tpu as pltpu
from jax.experimental.pallas import tpu_sc as plsc

N = 10000
E = 320000
D = 128
L = 5
G = 256
FEATS = 9
ATOM_VOCAB = 120
BOND_VOCAB = 5

NP = 10240
BLK = 512
NBLK = NP // BLK

NC = 2
NS = 16
NW = NC * NS
C = 80
CHUNKS_PW = 128
EPW = CHUNKS_PW * C
EP = NW * EPW
ROWS_PT = NP // NS
_SC_MESH = plsc.VectorSubcoreMesh(core_axis_name="c", subcore_axis_name="s")


def _edge_body(h_hbm, idx_hbm, bond_hbm, out_hbm,
               i0, i1, i2, i3, r0, r1, r2, r3, bond_v, aggr_sh,
               sem_i, sem_g, sem_s):
    c = lax.axis_index("c")
    s = lax.axis_index("s")
    wid = s * NC + c
    base = wid * CHUNKS_PW
    idxs = (i0, i1, i2, i3)
    rows = (r0, r1, r2, r3)

    zero = jnp.zeros((16,), jnp.float32)

    def _zrow(i, carry):
        for j in range(D // 16):
            r0[i, pl.ds(j * 16, 16)] = zero
        return carry

    lax.fori_loop(0, C, _zrow, 0)
    for k in range(ROWS_PT // C):
        pltpu.sync_copy(r0, aggr_sh.at[pl.ds(s * ROWS_PT + k * C, C)])
    plsc.subcore_barrier()

    pltpu.sync_copy(bond_hbm, bond_v)

    def _fetch_idx(k, t):
        pltpu.async_copy(idx_hbm.at[base + k], idxs[t], sem_i)

    def _gather(t):
        pltpu.async_copy(h_hbm.at[idxs[t].at[0]], rows[t], sem_g)

    def _drain_rows(sem, t):
        pltpu.make_async_copy(h_hbm.at[pl.ds(0, C)], rows[t], sem).wait()

    def _drain_idx(t):
        pltpu.make_async_copy(idx_hbm.at[0], idxs[t], sem_i).wait()

    def _compute(t):
        rv = rows[t]
        iv = idxs[t]

        def _grp(g2, cc):
            bvec = iv[2, pl.ds(g2 * 16, 16)]
            for i16 in range(16):
                b = bvec[i16]
                ro = g2 * 16 + i16
                for j in range(D // 16):
                    sl = pl.ds(j * 16, 16)
                    rv[ro, sl] = jnp.maximum(rv[ro, sl] + bond_v[b, sl], 0.0)
            return cc

        lax.fori_loop(0, C // 16, _grp, 0)

    def _scat(t):
        pltpu.async_copy(rows[t], aggr_sh.at[idxs[t].at[1]], sem_s, add=True)

    K = CHUNKS_PW
    _fetch_idx(0, 0)
    _fetch_idx(1, 1)
    _drain_idx(0)
    _gather(0)

    def _iter(g, cc):
        for j in range(4):
            kk = g * 4 + j
            t = j
            t1 = (j + 1) % 4
            t2 = (j + 2) % 4

            @pl.when(jnp.logical_and(kk >= 2, kk + 2 < K))
            def _():
                _drain_rows(sem_s, t2)

            @pl.when(kk + 2 < K)
            def _():
                _fetch_idx(kk + 2, t2)

            @pl.when(kk + 1 < K)
            def _():
                _drain_idx(t1)
                _gather(t1)

            _drain_rows(sem_g, t)
            _compute(t)
            _scat(t)
        return cc

    lax.fori_loop(0, K // 4, _iter, 0)
    for t in range(4):
        _drain_rows(sem_s, t)

    plsc.subcore_barrier()
    for k in range(ROWS_PT // C):
        start = s * ROWS_PT + k * C
        pltpu.sync_copy(aggr_sh.at[pl.ds(start, C)],
                        out_hbm.at[c, pl.ds(start, C)])


@jax.jit
def _edge_aggr(h, idx_p, bond_ext):
    fn = functools.partial(
        pl.kernel,
        mesh=_SC_MESH,
        out_type=jax.ShapeDtypeStruct((NC, NP, D), jnp.float32),
        scratch_types=[
            pltpu.VMEM((3, C), jnp.int32),
            pltpu.VMEM((3, C), jnp.int32),
            pltpu.VMEM((3, C), jnp.int32),
            pltpu.VMEM((3, C), jnp.int32),
            pltpu.VMEM((C, D), jnp.float32),
            pltpu.VMEM((C, D), jnp.float32),
            pltpu.VMEM((C, D), jnp.float32),
            pltpu.VMEM((C, D), jnp.float32),
            pltpu.VMEM((8, D), jnp.float32),
            pltpu.VMEM_SHARED((NP, D), jnp.float32),
            pltpu.SemaphoreType.DMA,
            pltpu.SemaphoreType.DMA,
            pltpu.SemaphoreType.DMA,
        ],
    )(_edge_body)
    return fn(h, idx_p, bond_ext)


def _enc_body(x_ref, emb_ref, o_ref):
    x = x_ref[...]
    acc = jnp.zeros((BLK, D), dtype=jnp.float32)
    for f in range(FEATS):
        col = x[:, f][:, None]
        iota = lax.broadcasted_iota(jnp.int32, (1, ATOM_VOCAB), 1)
        oh = (col == iota).astype(jnp.float32)
        acc = acc + jax.lax.dot(oh, emb_ref[f], preferred_element_type=jnp.float32)
    o_ref[...] = acc


@jax.jit
def _encoder(xp, atom_emb):
    return pl.pallas_call(
        _enc_body,
        grid=(NBLK,),
        in_specs=[
            pl.BlockSpec((BLK, FEATS), lambda i: (i, 0)),
            pl.BlockSpec((FEATS, ATOM_VOCAB, D), lambda i: (0, 0, 0)),
        ],
        out_specs=pl.BlockSpec((BLK, D), lambda i: (i, 0)),
        out_shape=jax.ShapeDtypeStruct((NP, D), jnp.float32),
    )(xp, atom_emb)


def _mlp_pool_body(h_ref, a_ref, bat_ref, eps_ref, w1_ref, b1_ref, w2_ref,
                   b2_ref, ho_ref, pool_ref):
    eps1 = 1.0 + eps_ref[0, 0]
    z = eps1 * h_ref[...] + a_ref[0] + a_ref[1]
    t = jax.lax.dot(z, w1_ref[...], preferred_element_type=jnp.float32) + b1_ref[...]
    t = jnp.maximum(t, 0.0)
    h2 = jax.lax.dot(t, w2_ref[...], preferred_element_type=jnp.float32) + b2_ref[...]
    ho_ref[...] = h2
    col = bat_ref[...][:, 0][:, None]
    iota = lax.broadcasted_iota(jnp.int32, (1, G), 1)
    oh = (col == iota).astype(jnp.float32)
    p = lax.dot_general(oh, h2, (((0,), (0,)), ((), ())),
                        preferred_element_type=jnp.float32)

    @pl.when(pl.program_id(0) == 0)
    def _init():
        pool_ref[...] = p

    @pl.when(pl.program_id(0) != 0)
    def _acc():
        pool_ref[...] = pool_ref[...] + p


@jax.jit
def _mlp_pool(h, aggr2, batp, eps_l, W1l, b1l, W2l, b2l):
    return pl.pallas_call(
        _mlp_pool_body,
        grid=(NBLK,),
        in_specs=[
            pl.BlockSpec((BLK, D), lambda i: (i, 0)),
            pl.BlockSpec((2, BLK, D), lambda i: (0, i, 0)),
            pl.BlockSpec((BLK, 1), lambda i: (i, 0)),
            pl.BlockSpec(memory_space=pltpu.SMEM),
            pl.BlockSpec((D, 2 * D), lambda i: (0, 0)),
            pl.BlockSpec((1, 2 * D), lambda i: (0, 0)),
            pl.BlockSpec((2 * D, D), lambda i: (0, 0)),
            pl.BlockSpec((1, D), lambda i: (0, 0)),
        ],
        out_specs=[
            pl.BlockSpec((BLK, D), lambda i: (i, 0)),
            pl.BlockSpec((G, D), lambda i: (0, 0)),
        ],
        out_shape=[
            jax.ShapeDtypeStruct((NP, D), jnp.float32),
            jax.ShapeDtypeStruct((G, D), jnp.float32),
        ],
    )(h, aggr2, batp, eps_l, W1l, b1l, W2l, b2l)


def kernel(x, edge_index, edge_attr, batch, atom_emb, bond_emb, W1, b1, W2, b2, eps):
    xp = jnp.zeros((NP, FEATS), jnp.int32).at[:N].set(x.astype(jnp.int32))
    batp = jnp.full((NP, 1), G, jnp.int32).at[:N, 0].set(batch.astype(jnp.int32))
    bidx = jnp.clip(edge_attr[:, 0], 0, BOND_VOCAB - 1).astype(jnp.int32)
    idx_p = jnp.full((3, EP), N, jnp.int32)
    idx_p = idx_p.at[0, :E].set(edge_index[0].astype(jnp.int32))
    idx_p = idx_p.at[1, :E].set(edge_index[1].astype(jnp.int32))
    idx_p = idx_p.at[2, :E].set(bidx).at[2, E:].set(BOND_VOCAB)
    idx_p = jnp.transpose(idx_p.reshape(3, NW * CHUNKS_PW, C), (1, 0, 2))
    bond_ext = jnp.zeros((8, D), jnp.float32).at[:BOND_VOCAB].set(bond_emb)

    h = _encoder(xp, atom_emb)
    outs = []
    pools = []
    for l in range(L):
        aggr2 = _edge_aggr(h, idx_p, bond_ext)
        eps_l = eps[l].reshape(1, 1)
        h, pool_l = _mlp_pool(h, aggr2, batp, eps_l, W1[l],
                              b1[l].reshape(1, -1), W2[l], b2[l].reshape(1, -1))
        outs.append(h[:N])
        pools.append(pool_l)
    node_embs = jnp.concatenate(outs, axis=-1)
    graph_embs = jnp.concatenate(pools, axis=-1)
    return graph_embs, node_embs
